# baseline (speedup 1.0000x reference)
import functools

import jax
import jax.numpy as jnp
from jax.experimental import pallas as pl

N = 50000
D = 64
STEP = 0.1
BLK = 1000


def _dense_body(x_ref, wl_ref, bl_ref, wab_ref, y_ref, ab_ref):
    x = x_ref[...]
    y_ref[...] = jnp.dot(x, wl_ref[...].T, preferred_element_type=jnp.float32) + bl_ref[...]
    ab_ref[...] = jnp.dot(x, wab_ref[...], preferred_element_type=jnp.float32)


def _dense(x, W_lin, b_lin, W_ab):
    grid = N // BLK
    return pl.pallas_call(
        _dense_body,
        grid=(grid,),
        in_specs=[
            pl.BlockSpec((BLK, D), lambda i: (i, 0)),
            pl.BlockSpec((D, D), lambda i: (0, 0)),
            pl.BlockSpec((D,), lambda i: (0,)),
            pl.BlockSpec((D, 128), lambda i: (0, 0)),
        ],
        out_specs=[
            pl.BlockSpec((BLK, D), lambda i: (i, 0)),
            pl.BlockSpec((BLK, 128), lambda i: (i, 0)),
        ],
        out_shape=[
            jax.ShapeDtypeStruct((N, D), jnp.float32),
            jax.ShapeDtypeStruct((N, 128), jnp.float32),
        ],
    )(x, W_lin, b_lin, W_ab)


def kernel(x, edge_index, W_sheaf, W_lin, b_lin):
    row = edge_index[0].astype(jnp.int32)
    col = edge_index[1].astype(jnp.int32)
    E = row.shape[0]
    half = E // 2

    W_ab = jnp.zeros((D, 128), jnp.float32)
    W_ab = W_ab.at[:, 0].set(W_sheaf[:D, 0]).at[:, 1].set(W_sheaf[D:, 0])
    y, ab = _dense(x, W_lin, b_lin, W_ab)
    a = ab[:, 0]
    b = ab[:, 1]

    maps = jnp.tanh(a[row] + b[col])
    rmaps = jnp.concatenate([maps[half:], maps[:half]])
    diag_maps = jax.ops.segment_sum(maps * maps, row, num_segments=N)
    d = jax.lax.rsqrt(diag_maps + 1.0)
    w = -(maps * rmaps) * d[row] * d[col]
    Ly = (diag_maps * d * d)[:, None] * y + jax.ops.segment_sum(
        w[:, None] * y[col], row, num_segments=N)
    return x - STEP * Ly


# speedup vs baseline: 12.7872x; 12.7872x over previous
import functools

import jax
import jax.numpy as jnp
from jax import lax
from jax.experimental import pallas as pl
from jax.experimental.pallas import tpu as pltpu
from jax.experimental.pallas import tpu_sc as plsc

N = 50000
D = 64
STEP = 0.1
NP = 50176
SLICE = NP // 16
DUMP = N
E = 800000
HALF = E // 2
EPAD = 819200
CA = 1600
CB = 800
Q = SLICE // 4

_mesh = plsc.VectorSubcoreMesh(core_axis_name="c", subcore_axis_name="s")
_f32 = jnp.float32
_sc_params = pltpu.CompilerParams(
    needs_layout_passes=False, use_tc_tiling_on_sc=False)


def _dense_body(x_ref, wlt_ref, bl_ref, ws_ref, y4_ref, ab_ref):
    x = x_ref[...]
    y = jnp.dot(x, wlt_ref[...], preferred_element_type=_f32) + bl_ref[...]
    for f in range(4):
        y4_ref[f] = y[:, f * 16:(f + 1) * 16]
    a = lax.dot_general(x, ws_ref[0], (((1,), (0,)), ((), ())),
                        preferred_element_type=_f32)
    b = lax.dot_general(x, ws_ref[1], (((1,), (0,)), ((), ())),
                        preferred_element_type=_f32)
    ab_ref[0] = a.reshape(8, 128)
    ab_ref[1] = b.reshape(8, 128)


def _phase1(x_p, wlt, bl2, ws2):
    return pl.pallas_call(
        _dense_body,
        grid=(NP // 1024,),
        in_specs=[
            pl.BlockSpec((1024, D), lambda i: (i, 0)),
            pl.BlockSpec((D, D), lambda i: (0, 0)),
            pl.BlockSpec((1, D), lambda i: (0, 0)),
            pl.BlockSpec((2, D), lambda i: (0, 0)),
        ],
        out_specs=[
            pl.BlockSpec((4, 1024, 16), lambda i: (0, i, 0)),
            pl.BlockSpec((2, 8, 128), lambda i: (0, i, 0)),
        ],
        out_shape=[
            jax.ShapeDtypeStruct((4, NP, 16), _f32),
            jax.ShapeDtypeStruct((2, 392, 128), _f32),
        ],
    )(x_p, wlt, bl2, ws2)


@functools.partial(
    pl.kernel,
    out_type=[jax.ShapeDtypeStruct((EPAD,), _f32),
              jax.ShapeDtypeStruct((NP,), _f32),
              jax.ShapeDtypeStruct((NP,), _f32)],
    mesh=_mesh,
    compiler_params=_sc_params,
    scratch_types=[
        pltpu.VMEM((NP,), _f32),
        pltpu.VMEM((NP,), _f32),
        pltpu.VMEM((CA,), jnp.int32),
        pltpu.VMEM((CA,), jnp.int32),
        pltpu.VMEM((CA,), _f32),
        pltpu.VMEM((CA,), _f32),
        pltpu.VMEM((SLICE,), _f32),
        pltpu.VMEM_SHARED((NP,), _f32),
    ],
)
def _sc_maps(row_h, col_h, a_h, b_h, maps_h, p0_h, p1_h,
             a_v, b_v, row_v, col_v, m_v, m2_v, z_v, acc_s):
    c = lax.axis_index("c")
    s = lax.axis_index("s")
    wid = c * 16 + s

    def zset(i, _):
        z_v[pl.ds(i * 16, 16)] = jnp.zeros((16,), _f32)
        return 0
    lax.fori_loop(0, SLICE // 16, zset, 0)
    pltpu.sync_copy(z_v, acc_s.at[pl.ds(s * SLICE, SLICE)])
    pltpu.sync_copy(a_h, a_v)
    pltpu.sync_copy(b_h, b_v)
    plsc.subcore_barrier()

    def chunk(j, _):
        base = (wid * 16 + j) * CA
        pltpu.sync_copy(row_h.at[pl.ds(base, CA)], row_v)
        pltpu.sync_copy(col_h.at[pl.ds(base, CA)], col_v)

        def grp(g, _):
            ri = row_v[pl.ds(g * 16, 16)]
            ci = col_v[pl.ds(g * 16, 16)]
            av = plsc.load_gather(a_v, [ri])
            bv = plsc.load_gather(b_v, [ci])
            z2 = (av + bv) * 2.0
            m = 1.0 - 2.0 / (jnp.exp(z2) + 1.0)
            m_v[pl.ds(g * 16, 16)] = m
            m2_v[pl.ds(g * 16, 16)] = m * m
            return 0
        lax.fori_loop(0, CA // 16, grp, 0)
        pltpu.sync_copy(m_v, maps_h.at[pl.ds(base, CA)])
        pltpu.sync_copy(m2_v, acc_s.at[row_v], add=True)
        return 0
    lax.fori_loop(0, 16, chunk, 0)
    plsc.subcore_barrier()

    @pl.when(c == 0)
    def _():
        pltpu.sync_copy(acc_s.at[pl.ds(s * SLICE, SLICE)],
                        p0_h.at[pl.ds(s * SLICE, SLICE)])

    @pl.when(c == 1)
    def _():
        pltpu.sync_copy(acc_s.at[pl.ds(s * SLICE, SLICE)],
                        p1_h.at[pl.ds(s * SLICE, SLICE)])


def _a2_body(p_ref, d_ref, dg_ref):
    sm = p_ref[0] + p_ref[1]
    d = lax.rsqrt(sm + 1.0)
    d_ref[...] = d
    dg_ref[...] = sm * d * d


def _a2(part3):
    return pl.pallas_call(
        _a2_body,
        grid=(49,),
        in_specs=[pl.BlockSpec((2, 8, 128), lambda i: (0, i, 0))],
        out_specs=[pl.BlockSpec((8, 128), lambda i: (i, 0)),
                   pl.BlockSpec((8, 128), lambda i: (i, 0))],
        out_shape=[jax.ShapeDtypeStruct((392, 128), _f32),
                   jax.ShapeDtypeStruct((392, 128), _f32)],
    )(part3)


@functools.partial(
    pl.kernel,
    out_type=jax.ShapeDtypeStruct((4, NP, 16), _f32),
    mesh=_mesh,
    compiler_params=_sc_params,
    scratch_types=[
        pltpu.VMEM((NP,), _f32),
        pltpu.VMEM((CB,), jnp.int32),
        pltpu.VMEM((CB,), jnp.int32),
        pltpu.VMEM((CB,), _f32),
        pltpu.VMEM((CB,), _f32),
        pltpu.VMEM((CB, 16), _f32),
        pltpu.VMEM((Q,), jnp.int32),
        pltpu.VMEM((Q,), _f32),
        pltpu.VMEM_SHARED((NP, 16), _f32),
        pltpu.SemaphoreType.DMA,
    ],
)
def _sc_spmm(row_h, col_h, maps_h, d_h, dg_h, y4_h, acc4_h,
             d_v, row_v, col_v, m_v, rm_v, rows_v, ix_v, dg_v, acc_s, sem):
    c = lax.axis_index("c")
    s = lax.axis_index("s")
    z16 = jnp.zeros((16,), _f32)

    pltpu.sync_copy(d_h, d_v)

    for p in range(2):
        fid = 2 * p + c

        def zgrp(g, _):
            ei = lax.iota(jnp.int32, 16) + g * 16
            for k in range(16):
                kk = jnp.full((16,), k, jnp.int32)
                plsc.store_scatter(rows_v, [ei, kk], z16)
            return 0
        lax.fori_loop(0, CB // 16, zgrp, 0)
        for q in range(4):
            pltpu.sync_copy(rows_v.at[pl.ds(0, Q)],
                            acc_s.at[pl.ds(s * SLICE + q * Q, Q)])
        plsc.subcore_barrier()

        def chunk(j, _):
            base = (s * 64 + j) * CB
            pltpu.sync_copy(row_h.at[pl.ds(base, CB)], row_v)
            pltpu.sync_copy(col_h.at[pl.ds(base, CB)], col_v)
            cp = pltpu.async_copy(y4_h.at[fid].at[col_v], rows_v, sem)
            rmb = jnp.where(base < HALF, base + HALF, base - HALF)
            pltpu.sync_copy(maps_h.at[pl.ds(base, CB)], m_v)
            pltpu.sync_copy(maps_h.at[pl.ds(rmb, CB)], rm_v)
            cp.wait()

            def grp(g, _):
                gb = g * 16
                m = m_v[pl.ds(gb, 16)]
                rm = rm_v[pl.ds(gb, 16)]
                ri = row_v[pl.ds(gb, 16)]
                ci = col_v[pl.ds(gb, 16)]
                dr = plsc.load_gather(d_v, [ri])
                dc = plsc.load_gather(d_v, [ci])
                w = -(m * rm) * dr * dc
                ei = lax.iota(jnp.int32, 16) + gb
                for k in range(16):
                    kk = jnp.full((16,), k, jnp.int32)
                    v = plsc.load_gather(rows_v, [ei, kk])
                    plsc.store_scatter(rows_v, [ei, kk], v * w)
                return 0
            lax.fori_loop(0, CB // 16, grp, 0)
            pltpu.sync_copy(rows_v, acc_s.at[row_v], add=True)
            return 0
        lax.fori_loop(0, 64, chunk, 0)

        for q in range(4):
            nb = s * SLICE + q * Q
            pltpu.sync_copy(y4_h.at[fid].at[pl.ds(nb, Q)],
                            rows_v.at[pl.ds(0, Q)])
            pltpu.sync_copy(dg_h.at[pl.ds(nb, Q)], dg_v)

            def dgrp(g, _):
                gb = g * 16
                dgv = dg_v[pl.ds(gb, 16)]
                ei = lax.iota(jnp.int32, 16) + gb
                ix_v[pl.ds(gb, 16)] = ei + nb
                for k in range(16):
                    kk = jnp.full((16,), k, jnp.int32)
                    v = plsc.load_gather(rows_v, [ei, kk])
                    plsc.store_scatter(rows_v, [ei, kk], v * dgv)
                return 0
            lax.fori_loop(0, Q // 16, dgrp, 0)
            pltpu.sync_copy(rows_v.at[pl.ds(0, Q)], acc_s.at[ix_v], add=True)

        plsc.subcore_barrier()
        pltpu.sync_copy(acc_s.at[pl.ds(s * SLICE, SLICE)],
                        acc4_h.at[fid, pl.ds(s * SLICE, SLICE)])
        plsc.subcore_barrier()


def _comb_body(x_ref, acc_ref, o_ref):
    acc = jnp.concatenate([acc_ref[f] for f in range(4)], axis=-1)
    o_ref[...] = x_ref[...] - STEP * acc


def _combine(x, acc4):
    return pl.pallas_call(
        _comb_body,
        grid=(25,),
        in_specs=[
            pl.BlockSpec((2000, D), lambda i: (i, 0)),
            pl.BlockSpec((4, 2000, 16), lambda i: (0, i, 0)),
        ],
        out_specs=pl.BlockSpec((2000, D), lambda i: (i, 0)),
        out_shape=jax.ShapeDtypeStruct((N, D), _f32),
    )(x, acc4)


def kernel(x, edge_index, W_sheaf, W_lin, b_lin):
    row = edge_index[0].astype(jnp.int32)
    col = edge_index[1].astype(jnp.int32)
    pad_i = jnp.full((EPAD - E,), DUMP, jnp.int32)
    row_p = jnp.concatenate([row, pad_i])
    col_p = jnp.concatenate([col, pad_i])
    x_p = jnp.zeros((NP, D), _f32).at[:N].set(x)
    wlt = W_lin.T
    bl2 = b_lin.reshape(1, D)
    ws2 = W_sheaf[:, 0].reshape(2, D)

    y4, ab5 = _phase1(x_p, wlt, bl2, ws2)
    maps, p0, p1 = _sc_maps(row_p, col_p, ab5[0].reshape(NP), ab5[1].reshape(NP))
    d4, dg4 = _a2(jnp.stack([p0, p1]).reshape(2, 392, 128))
    acc4 = _sc_spmm(row_p, col_p, maps, d4.reshape(NP), dg4.reshape(NP), y4)
    return _combine(x, acc4)


# speedup vs baseline: 23.7963x; 1.8609x over previous
import functools

import jax
import jax.numpy as jnp
from jax import lax
from jax.experimental import pallas as pl
from jax.experimental.pallas import tpu as pltpu
from jax.experimental.pallas import tpu_sc as plsc

N = 50000
D = 64
STEP = 0.1
NP = 50176
SLICE = NP // 16
DUMP = N
E = 800000
HALF = E // 2
EPAD = 819200
CA = 1600
CB = 800
Q = SLICE // 4

_mesh = plsc.VectorSubcoreMesh(core_axis_name="c", subcore_axis_name="s")
_f32 = jnp.float32
_sc_params = pltpu.CompilerParams(
    needs_layout_passes=False, use_tc_tiling_on_sc=False)


def _dense_body(x_ref, wlt_ref, bl_ref, ws_ref, y2_ref, ab_ref):
    x = x_ref[...]
    y = jnp.dot(x, wlt_ref[...], preferred_element_type=_f32) + bl_ref[...]
    yb = y.astype(jnp.bfloat16)
    y2_ref[0] = yb[:, :32]
    y2_ref[1] = yb[:, 32:]
    a = lax.dot_general(x, ws_ref[0], (((1,), (0,)), ((), ())),
                        preferred_element_type=_f32)
    b = lax.dot_general(x, ws_ref[1], (((1,), (0,)), ((), ())),
                        preferred_element_type=_f32)
    ab_ref[0] = a.reshape(8, 128)
    ab_ref[1] = b.reshape(8, 128)


def _phase1(x_p, wlt, bl2, ws2):
    return pl.pallas_call(
        _dense_body,
        grid=(NP // 1024,),
        in_specs=[
            pl.BlockSpec((1024, D), lambda i: (i, 0)),
            pl.BlockSpec((D, D), lambda i: (0, 0)),
            pl.BlockSpec((1, D), lambda i: (0, 0)),
            pl.BlockSpec((2, D), lambda i: (0, 0)),
        ],
        out_specs=[
            pl.BlockSpec((2, 1024, 32), lambda i: (0, i, 0)),
            pl.BlockSpec((2, 8, 128), lambda i: (0, i, 0)),
        ],
        out_shape=[
            jax.ShapeDtypeStruct((2, NP, 32), jnp.bfloat16),
            jax.ShapeDtypeStruct((2, 392, 128), _f32),
        ],
    )(x_p, wlt, bl2, ws2)


@functools.partial(
    pl.kernel,
    out_type=[jax.ShapeDtypeStruct((EPAD,), _f32),
              jax.ShapeDtypeStruct((NP,), _f32),
              jax.ShapeDtypeStruct((NP,), _f32)],
    mesh=_mesh,
    compiler_params=_sc_params,
    scratch_types=[
        pltpu.VMEM((NP,), _f32),
        pltpu.VMEM((NP,), _f32),
        pltpu.VMEM((CA,), jnp.int32),
        pltpu.VMEM((CA,), jnp.int32),
        pltpu.VMEM((CA,), _f32),
        pltpu.VMEM((CA,), _f32),
        pltpu.VMEM((SLICE,), _f32),
        pltpu.VMEM_SHARED((NP,), _f32),
    ],
)
def _sc_maps(row_h, col_h, a_h, b_h, maps_h, p0_h, p1_h,
             a_v, b_v, row_v, col_v, m_v, m2_v, z_v, acc_s):
    c = lax.axis_index("c")
    s = lax.axis_index("s")
    wid = c * 16 + s

    def zset(i, _):
        z_v[pl.ds(i * 16, 16)] = jnp.zeros((16,), _f32)
        return 0
    lax.fori_loop(0, SLICE // 16, zset, 0)
    pltpu.sync_copy(z_v, acc_s.at[pl.ds(s * SLICE, SLICE)])
    pltpu.sync_copy(a_h, a_v)
    pltpu.sync_copy(b_h, b_v)
    plsc.subcore_barrier()

    def chunk(j, _):
        base = (wid * 16 + j) * CA
        pltpu.sync_copy(row_h.at[pl.ds(base, CA)], row_v)
        pltpu.sync_copy(col_h.at[pl.ds(base, CA)], col_v)

        def grp(g, _):
            ri = row_v[pl.ds(g * 16, 16)]
            ci = col_v[pl.ds(g * 16, 16)]
            av = plsc.load_gather(a_v, [ri])
            bv = plsc.load_gather(b_v, [ci])
            z2 = (av + bv) * 2.0
            m = 1.0 - 2.0 / (jnp.exp(z2) + 1.0)
            m_v[pl.ds(g * 16, 16)] = m
            m2_v[pl.ds(g * 16, 16)] = m * m
            return 0
        lax.fori_loop(0, CA // 16, grp, 0)
        pltpu.sync_copy(m_v, maps_h.at[pl.ds(base, CA)])
        pltpu.sync_copy(m2_v, acc_s.at[row_v], add=True)
        return 0
    lax.fori_loop(0, 16, chunk, 0)
    plsc.subcore_barrier()

    @pl.when(c == 0)
    def _():
        pltpu.sync_copy(acc_s.at[pl.ds(s * SLICE, SLICE)],
                        p0_h.at[pl.ds(s * SLICE, SLICE)])

    @pl.when(c == 1)
    def _():
        pltpu.sync_copy(acc_s.at[pl.ds(s * SLICE, SLICE)],
                        p1_h.at[pl.ds(s * SLICE, SLICE)])


def _a2_body(p0_ref, p1_ref, y_ref, d_ref, dy_ref):
    sm = p0_ref[...] + p1_ref[...]
    d = lax.rsqrt(sm + 1.0)
    d_ref[...] = d
    dg = sm * d * d
    dy_ref[0] = (dg * y_ref[0].astype(_f32)).astype(jnp.bfloat16)
    dy_ref[1] = (dg * y_ref[1].astype(_f32)).astype(jnp.bfloat16)


def _a2(p0, p1, y2):
    return pl.pallas_call(
        _a2_body,
        grid=(49,),
        in_specs=[pl.BlockSpec((1024, 1), lambda i: (i, 0)),
                  pl.BlockSpec((1024, 1), lambda i: (i, 0)),
                  pl.BlockSpec((2, 1024, 32), lambda i: (0, i, 0))],
        out_specs=[pl.BlockSpec((1024, 1), lambda i: (i, 0)),
                   pl.BlockSpec((2, 1024, 32), lambda i: (0, i, 0))],
        out_shape=[jax.ShapeDtypeStruct((NP, 1), _f32),
                   jax.ShapeDtypeStruct((2, NP, 32), jnp.bfloat16)],
    )(p0, p1, y2)


NCH = 64
_bf16 = jnp.bfloat16


@functools.partial(
    pl.kernel,
    out_type=jax.ShapeDtypeStruct((2, NP, 32), jnp.bfloat16),
    mesh=_mesh,
    compiler_params=_sc_params,
    scratch_types=[
        pltpu.VMEM((NP // 2,), jnp.int32),
        pltpu.VMEM((2, 2, CB), jnp.int32),
        pltpu.VMEM((CB,), _f32),
        pltpu.VMEM((CB,), _f32),
        pltpu.VMEM((CB,), _f32),
        pltpu.VMEM((2, CB, 32), jnp.bfloat16),
        pltpu.VMEM_SHARED((NP, 32), jnp.bfloat16),
        pltpu.SemaphoreType.DMA,
    ],
)
def _sc_spmm(row_h, col_h, maps_h, dpk_h, y2_h, acc2_h,
             dp_v, rc_v, m_v, rm_v, w_v, rows_v, acc_s, sem):
    c = lax.axis_index("c")
    s = lax.axis_index("s")

    zb = jnp.zeros((32,), jnp.bfloat16)

    def zrow(i, _):
        rows_v[0, i, pl.ds(0, 32)] = zb
        return 0
    lax.fori_loop(0, Q, zrow, 0)
    for q in range(4):
        pltpu.sync_copy(rows_v.at[0].at[pl.ds(0, Q)],
                        acc_s.at[pl.ds(s * SLICE + q * Q, Q)])
    pltpu.sync_copy(dpk_h, dp_v)
    plsc.subcore_barrier()

    tbase = s * NCH

    def issue(k, sl):
        base = (tbase + k) * CB
        pltpu.sync_copy(row_h.at[pl.ds(base, CB)], rc_v.at[sl, 0])
        pltpu.sync_copy(col_h.at[pl.ds(base, CB)], rc_v.at[sl, 1])
        pltpu.async_copy(y2_h.at[c].at[rc_v.at[sl, 1]], rows_v.at[sl], sem)

    def drain(sl):
        pltpu.make_async_copy(y2_h.at[c].at[pl.ds(0, CB)], rows_v.at[sl],
                              sem).wait()

    issue(0, 0)

    def phase(k, _):
        sl = k & 1

        @pl.when(k + 1 < NCH)
        def _():
            issue(k + 1, 1 - sl)

        base = (tbase + k) * CB
        rmb = jnp.where(base < HALF, base + HALF, base - HALF)
        pltpu.sync_copy(maps_h.at[pl.ds(base, CB)], m_v)
        pltpu.sync_copy(maps_h.at[pl.ds(rmb, CB)], rm_v)

        def unpack_d(idx):
            v = plsc.load_gather(dp_v, [jnp.right_shift(idx, 1)])
            half = jnp.where((idx & 1) == 1,
                             lax.shift_right_logical(v, 16), v & 0xFFFF)
            return plsc.bitcast(half << 16, _f32)

        def grp(g, _):
            gb = g * 16
            m = m_v[pl.ds(gb, 16)]
            rm = rm_v[pl.ds(gb, 16)]
            ri = rc_v[sl, 0, pl.ds(gb, 16)]
            ci = rc_v[sl, 1, pl.ds(gb, 16)]
            dr = unpack_d(ri)
            dc = unpack_d(ci)
            w_v[pl.ds(gb, 16)] = -(m * rm) * dr * dc
            return 0
        lax.fori_loop(0, CB // 16, grp, 0)
        drain(sl)

        def scale(i, _):
            wi = plsc.load_gather(w_v, [jnp.full((16,), i, jnp.int32)])
            wpk = plsc.pack(wi, wi, format=plsc.PackFormat.INTERLEAVED)
            rows_v[sl, i, pl.ds(0, 32)] = rows_v[sl, i, pl.ds(0, 32)] * wpk
            return 0
        lax.fori_loop(0, CB, scale, 0)
        pltpu.sync_copy(rows_v.at[sl], acc_s.at[rc_v.at[sl, 0]], add=True)
        return 0
    lax.fori_loop(0, NCH, phase, 0)

    plsc.subcore_barrier()
    pltpu.sync_copy(acc_s.at[pl.ds(s * SLICE, SLICE)],
                    acc2_h.at[c, pl.ds(s * SLICE, SLICE)])


def _comb_body(x_ref, acc_ref, dy_ref, o_ref):
    acc = jnp.concatenate([acc_ref[0], acc_ref[1]], axis=-1).astype(_f32)
    dy = jnp.concatenate([dy_ref[0], dy_ref[1]], axis=-1).astype(_f32)
    o_ref[...] = x_ref[...] - STEP * (acc + dy)


def _combine(x, acc2, dy2):
    return pl.pallas_call(
        _comb_body,
        grid=(25,),
        in_specs=[
            pl.BlockSpec((2000, D), lambda i: (i, 0)),
            pl.BlockSpec((2, 2000, 32), lambda i: (0, i, 0)),
            pl.BlockSpec((2, 2000, 32), lambda i: (0, i, 0)),
        ],
        out_specs=pl.BlockSpec((2000, D), lambda i: (i, 0)),
        out_shape=jax.ShapeDtypeStruct((N, D), _f32),
    )(x, acc2, dy2)


def kernel(x, edge_index, W_sheaf, W_lin, b_lin):
    row = edge_index[0].astype(jnp.int32)
    col = edge_index[1].astype(jnp.int32)
    pad_i = jnp.full((EPAD - E,), DUMP, jnp.int32)
    row_p = jnp.concatenate([row, pad_i])
    col_p = jnp.concatenate([col, pad_i])
    x_p = jnp.zeros((NP, D), _f32).at[:N].set(x)
    wlt = W_lin.T
    bl2 = b_lin.reshape(1, D)
    ws2 = W_sheaf[:, 0].reshape(2, D)

    y2, ab5 = _phase1(x_p, wlt, bl2, ws2)
    maps, p0, p1 = _sc_maps(row_p, col_p, ab5[0].reshape(NP), ab5[1].reshape(NP))
    d4, dy2 = _a2(p0.reshape(NP, 1), p1.reshape(NP, 1), y2)
    dbf = d4.reshape(NP).astype(jnp.bfloat16)
    dpk = jax.lax.bitcast_convert_type(dbf.reshape(NP // 2, 2), jnp.int32)
    acc2 = _sc_spmm(row_p, col_p, maps, dpk, y2)
    return _combine(x, acc2, dy2)


# speedup vs baseline: 25.1511x; 1.0569x over previous
import functools

import jax
import jax.numpy as jnp
from jax import lax
from jax.experimental import pallas as pl
from jax.experimental.pallas import tpu as pltpu
from jax.experimental.pallas import tpu_sc as plsc

N = 50000
D = 64
STEP = 0.1
NP = 50176
SLICE = NP // 16
DUMP = N
E = 800000
HALF = E // 2
EPAD = 819200
CA = 1600
CB = 800
Q = SLICE // 4

_mesh = plsc.VectorSubcoreMesh(core_axis_name="c", subcore_axis_name="s")
_f32 = jnp.float32
_sc_params = pltpu.CompilerParams(
    needs_layout_passes=False, use_tc_tiling_on_sc=False)


def _dense_body(x_ref, wlt_ref, bl_ref, ws_ref, y2_ref, ab_ref):
    x = x_ref[...]
    y = jnp.dot(x, wlt_ref[...], preferred_element_type=_f32) + bl_ref[...]
    yb = y.astype(jnp.bfloat16)
    y2_ref[0] = yb[:, :32]
    y2_ref[1] = yb[:, 32:]
    a = lax.dot_general(x, ws_ref[0], (((1,), (0,)), ((), ())),
                        preferred_element_type=_f32)
    b = lax.dot_general(x, ws_ref[1], (((1,), (0,)), ((), ())),
                        preferred_element_type=_f32)
    ab_ref[0] = a.reshape(8, 128)
    ab_ref[1] = b.reshape(8, 128)


def _phase1(x_p, wlt, bl2, ws2):
    return pl.pallas_call(
        _dense_body,
        grid=(NP // 1024,),
        in_specs=[
            pl.BlockSpec((1024, D), lambda i: (i, 0)),
            pl.BlockSpec((D, D), lambda i: (0, 0)),
            pl.BlockSpec((1, D), lambda i: (0, 0)),
            pl.BlockSpec((2, D), lambda i: (0, 0)),
        ],
        out_specs=[
            pl.BlockSpec((2, 1024, 32), lambda i: (0, i, 0)),
            pl.BlockSpec((2, 8, 128), lambda i: (0, i, 0)),
        ],
        out_shape=[
            jax.ShapeDtypeStruct((2, NP, 32), jnp.bfloat16),
            jax.ShapeDtypeStruct((2, 392, 128), _f32),
        ],
    )(x_p, wlt, bl2, ws2)


@functools.partial(
    pl.kernel,
    out_type=[jax.ShapeDtypeStruct((EPAD,), _f32),
              jax.ShapeDtypeStruct((NP,), _f32),
              jax.ShapeDtypeStruct((NP,), _f32)],
    mesh=_mesh,
    compiler_params=_sc_params,
    scratch_types=[
        pltpu.VMEM((NP,), _f32),
        pltpu.VMEM((NP,), _f32),
        pltpu.VMEM((CA,), jnp.int32),
        pltpu.VMEM((CA,), jnp.int32),
        pltpu.VMEM((CA,), _f32),
        pltpu.VMEM((CA,), _f32),
        pltpu.VMEM((SLICE,), _f32),
        pltpu.VMEM_SHARED((NP,), _f32),
    ],
)
def _sc_maps(row_h, col_h, a_h, b_h, maps_h, p0_h, p1_h,
             a_v, b_v, row_v, col_v, m_v, m2_v, z_v, acc_s):
    c = lax.axis_index("c")
    s = lax.axis_index("s")
    wid = c * 16 + s

    def zset(i, _):
        z_v[pl.ds(i * 16, 16)] = jnp.zeros((16,), _f32)
        return 0
    lax.fori_loop(0, SLICE // 16, zset, 0)
    pltpu.sync_copy(z_v, acc_s.at[pl.ds(s * SLICE, SLICE)])
    pltpu.sync_copy(a_h, a_v)
    pltpu.sync_copy(b_h, b_v)
    plsc.subcore_barrier()

    def chunk(j, _):
        base = (wid * 16 + j) * CA
        pltpu.sync_copy(row_h.at[pl.ds(base, CA)], row_v)
        pltpu.sync_copy(col_h.at[pl.ds(base, CA)], col_v)

        def grp(g, _):
            ri = row_v[pl.ds(g * 16, 16)]
            ci = col_v[pl.ds(g * 16, 16)]
            av = plsc.load_gather(a_v, [ri])
            bv = plsc.load_gather(b_v, [ci])
            z2 = (av + bv) * 2.0
            m = 1.0 - 2.0 / (jnp.exp(z2) + 1.0)
            m_v[pl.ds(g * 16, 16)] = m
            m2_v[pl.ds(g * 16, 16)] = m * m
            return 0
        lax.fori_loop(0, CA // 16, grp, 0)
        pltpu.sync_copy(m_v, maps_h.at[pl.ds(base, CA)])
        pltpu.sync_copy(m2_v, acc_s.at[row_v], add=True)
        return 0
    lax.fori_loop(0, 16, chunk, 0)
    plsc.subcore_barrier()

    @pl.when(c == 0)
    def _():
        pltpu.sync_copy(acc_s.at[pl.ds(s * SLICE, SLICE)],
                        p0_h.at[pl.ds(s * SLICE, SLICE)])

    @pl.when(c == 1)
    def _():
        pltpu.sync_copy(acc_s.at[pl.ds(s * SLICE, SLICE)],
                        p1_h.at[pl.ds(s * SLICE, SLICE)])


def _a2_body(p0_ref, p1_ref, y_ref, d_ref, dy_ref):
    sm = p0_ref[...] + p1_ref[...]
    d = lax.rsqrt(sm + 1.0)
    d_ref[...] = d
    dg = sm * d * d
    dy_ref[0] = (dg * y_ref[0].astype(_f32)).astype(jnp.bfloat16)
    dy_ref[1] = (dg * y_ref[1].astype(_f32)).astype(jnp.bfloat16)


def _a2(p0, p1, y2):
    return pl.pallas_call(
        _a2_body,
        grid=(49,),
        in_specs=[pl.BlockSpec((1024, 1), lambda i: (i, 0)),
                  pl.BlockSpec((1024, 1), lambda i: (i, 0)),
                  pl.BlockSpec((2, 1024, 32), lambda i: (0, i, 0))],
        out_specs=[pl.BlockSpec((1024, 1), lambda i: (i, 0)),
                   pl.BlockSpec((2, 1024, 32), lambda i: (0, i, 0))],
        out_shape=[jax.ShapeDtypeStruct((NP, 1), _f32),
                   jax.ShapeDtypeStruct((2, NP, 32), jnp.bfloat16)],
    )(p0, p1, y2)


NCH = 64
_bf16 = jnp.bfloat16


@functools.partial(
    pl.kernel,
    out_type=jax.ShapeDtypeStruct((2, NP, 32), jnp.bfloat16),
    mesh=_mesh,
    compiler_params=_sc_params,
    scratch_types=[
        pltpu.VMEM((NP // 2,), jnp.int32),
        pltpu.VMEM((2, 2, CB), jnp.int32),
        pltpu.VMEM((CB,), _f32),
        pltpu.VMEM((CB,), _f32),
        pltpu.VMEM((CB,), _f32),
        pltpu.VMEM((2, CB, 32), jnp.bfloat16),
        pltpu.VMEM_SHARED((NP, 32), jnp.bfloat16),
        pltpu.SemaphoreType.DMA,
    ],
)
def _sc_spmm(row_h, col_h, maps_h, dpk_h, y2_h, acc2_h,
             dp_v, rc_v, m_v, rm_v, w_v, rows_v, acc_s, sem):
    c = lax.axis_index("c")
    s = lax.axis_index("s")

    zb = jnp.zeros((32,), jnp.bfloat16)

    def zrow(i, _):
        rows_v[0, i, pl.ds(0, 32)] = zb
        return 0
    lax.fori_loop(0, Q, zrow, 0)
    for q in range(4):
        pltpu.sync_copy(rows_v.at[0].at[pl.ds(0, Q)],
                        acc_s.at[pl.ds(s * SLICE + q * Q, Q)])
    pltpu.sync_copy(dpk_h, dp_v)
    plsc.subcore_barrier()

    tbase = s * NCH

    def issue(k, sl):
        base = (tbase + k) * CB
        pltpu.sync_copy(row_h.at[pl.ds(base, CB)], rc_v.at[sl, 0])
        pltpu.sync_copy(col_h.at[pl.ds(base, CB)], rc_v.at[sl, 1])
        pltpu.async_copy(y2_h.at[c].at[rc_v.at[sl, 1]], rows_v.at[sl], sem)

    def drain(sl):
        pltpu.make_async_copy(y2_h.at[c].at[pl.ds(0, CB)], rows_v.at[sl],
                              sem).wait()

    issue(0, 0)

    def phase(k, _):
        sl = k & 1

        @pl.when(k + 1 < NCH)
        def _():
            issue(k + 1, 1 - sl)

        base = (tbase + k) * CB
        rmb = jnp.where(base < HALF, base + HALF, base - HALF)
        pltpu.sync_copy(maps_h.at[pl.ds(base, CB)], m_v)
        pltpu.sync_copy(maps_h.at[pl.ds(rmb, CB)], rm_v)

        def unpack_d(idx):
            v = plsc.load_gather(dp_v, [jnp.right_shift(idx, 1)])
            half = jnp.where((idx & 1) == 1,
                             lax.shift_right_logical(v, 16), v & 0xFFFF)
            return plsc.bitcast(half << 16, _f32)

        def grp(g, _):
            gb = g * 16
            m = m_v[pl.ds(gb, 16)]
            rm = rm_v[pl.ds(gb, 16)]
            ri = rc_v[sl, 0, pl.ds(gb, 16)]
            ci = rc_v[sl, 1, pl.ds(gb, 16)]
            dr = unpack_d(ri)
            dc = unpack_d(ci)
            w_v[pl.ds(gb, 16)] = -(m * rm) * dr * dc
            return 0
        lax.fori_loop(0, CB // 16, grp, 0)
        drain(sl)

        def scale(i, _):
            for u in range(4):
                r = i * 4 + u
                wi = plsc.load_gather(w_v, [jnp.full((16,), r, jnp.int32)])
                wpk = plsc.pack(wi, wi, format=plsc.PackFormat.INTERLEAVED)
                rows_v[sl, r, pl.ds(0, 32)] = (
                    rows_v[sl, r, pl.ds(0, 32)] * wpk)
            return 0
        lax.fori_loop(0, CB // 4, scale, 0)
        pltpu.sync_copy(rows_v.at[sl], acc_s.at[rc_v.at[sl, 0]], add=True)
        return 0
    lax.fori_loop(0, NCH, phase, 0)

    plsc.subcore_barrier()
    pltpu.sync_copy(acc_s.at[pl.ds(s * SLICE, SLICE)],
                    acc2_h.at[c, pl.ds(s * SLICE, SLICE)])


def _comb_body(x_ref, acc_ref, dy_ref, o_ref):
    acc = jnp.concatenate([acc_ref[0], acc_ref[1]], axis=-1).astype(_f32)
    dy = jnp.concatenate([dy_ref[0], dy_ref[1]], axis=-1).astype(_f32)
    o_ref[...] = x_ref[...] - STEP * (acc + dy)


def _combine(x, acc2, dy2):
    return pl.pallas_call(
        _comb_body,
        grid=(25,),
        in_specs=[
            pl.BlockSpec((2000, D), lambda i: (i, 0)),
            pl.BlockSpec((2, 2000, 32), lambda i: (0, i, 0)),
            pl.BlockSpec((2, 2000, 32), lambda i: (0, i, 0)),
        ],
        out_specs=pl.BlockSpec((2000, D), lambda i: (i, 0)),
        out_shape=jax.ShapeDtypeStruct((N, D), _f32),
    )(x, acc2, dy2)


def kernel(x, edge_index, W_sheaf, W_lin, b_lin):
    row = edge_index[0].astype(jnp.int32)
    col = edge_index[1].astype(jnp.int32)
    pad_i = jnp.full((EPAD - E,), DUMP, jnp.int32)
    row_p = jnp.concatenate([row, pad_i])
    col_p = jnp.concatenate([col, pad_i])
    x_p = jnp.zeros((NP, D), _f32).at[:N].set(x)
    wlt = W_lin.T
    bl2 = b_lin.reshape(1, D)
    ws2 = W_sheaf[:, 0].reshape(2, D)

    y2, ab5 = _phase1(x_p, wlt, bl2, ws2)
    maps, p0, p1 = _sc_maps(row_p, col_p, ab5[0].reshape(NP), ab5[1].reshape(NP))
    d4, dy2 = _a2(p0.reshape(NP, 1), p1.reshape(NP, 1), y2)
    dbf = d4.reshape(NP).astype(jnp.bfloat16)
    dpk = jax.lax.bitcast_convert_type(dbf.reshape(NP // 2, 2), jnp.int32)
    acc2 = _sc_spmm(row_p, col_p, maps, dpk, y2)
    return _combine(x, acc2, dy2)
